# baseline (speedup 1.0000x reference)
import functools

import jax
import jax.numpy as jnp
from jax import lax
from jax.experimental import pallas as pl
from jax.experimental.pallas import tpu as pltpu
from jax.experimental.pallas import tpu_sc as plsc

_K = 8192
_D = 256
_BM = 256
_B1 = 2736
_B2 = 5472
_A0 = 2688
_A1 = 5376
_BETA = 0.25


def _rowmin(a):
    return jnp.min(a, axis=1, keepdims=True)


def _vq_block(x_ref, w_ref, x2_ref, w2_ref, sid_ref, enc_ref, idx_ref,
              dsel_ref):
    x = x_ref[...]
    wb = w_ref[...]
    xb2 = x.astype(jnp.bfloat16) * jnp.bfloat16(2.0)
    t2 = jax.lax.dot_general(
        xb2, wb, (((1,), (1,)), ((), ())),
        preferred_element_type=jnp.float32)
    d = (x2_ref[...] + w2_ref[...]) - t2
    inf = jnp.float32(jnp.inf)
    sA = d[:, _A0:_A0 + 128]
    sB = d[:, _A1:_A1 + 128]
    lane = jax.lax.broadcasted_iota(jnp.int32, (_BM, 128), 1)
    mA = lane < (_B1 - _A0)
    mB = lane < (_B2 - _A1)
    m0 = jnp.minimum(_rowmin(d[:, :_A0]), _rowmin(jnp.where(mA, sA, inf)))
    m1 = jnp.minimum(
        jnp.minimum(_rowmin(jnp.where(mA, inf, sA)), _rowmin(d[:, _A0 + 128:_A1])),
        _rowmin(jnp.where(mB, sB, inf)))
    m2 = jnp.minimum(_rowmin(jnp.where(mB, inf, sB)), _rowmin(d[:, _A1 + 128:]))
    b0 = m0.astype(jnp.bfloat16).astype(jnp.float32)
    win1 = m1 < b0
    v1 = jnp.where(win1, m1, m0)
    b1 = v1.astype(jnp.bfloat16).astype(jnp.float32)
    win2 = m2 < b1
    dsel = jnp.where(win2, m2, v1)
    seg_sel = jnp.where(win2, 2, jnp.where(win1, 1, 0))
    cols = jax.lax.broadcasted_iota(jnp.int32, d.shape, 1)
    colsf = cols.astype(jnp.float32)
    pick = (d == dsel) & (sid_ref[...] == seg_sel)
    idxf = jnp.min(jnp.where(pick, colsf, jnp.float32(_K)), axis=1,
                   keepdims=True)
    idx = idxf.astype(jnp.int32)
    enc = (colsf == idxf).astype(jnp.float32)
    enc_ref[...] = enc
    idx_ref[...] = idx
    dsel_ref[...] = dsel


def _sc_gather(table, idx):
    info = plsc.get_sparse_core_info()
    nw = info.num_cores * info.num_subcores
    nb = idx.shape[0]
    dm = table.shape[1]
    b_per_w = nb // nw
    ch = 64
    nit = b_per_w // ch
    mesh = plsc.VectorSubcoreMesh(core_axis_name="c", subcore_axis_name="s")

    @functools.partial(
        pl.kernel, mesh=mesh,
        out_type=jax.ShapeDtypeStruct((nb, dm), jnp.float32),
        scratch_types=[
            pltpu.VMEM((ch,), jnp.int32),
            pltpu.VMEM((ch, dm), jnp.float32),
            pltpu.SemaphoreType.DMA,
        ],
    )
    def gk(table_hbm, idx_hbm, out_hbm, idx_v, rows_v, sem):
        wid = lax.axis_index("s") * info.num_cores + lax.axis_index("c")
        base = wid * b_per_w

        def body(i, carry):
            b = base + i * ch
            pltpu.sync_copy(idx_hbm.at[pl.ds(b, ch)], idx_v)
            pltpu.async_copy(table_hbm.at[idx_v], rows_v, sem).wait()
            pltpu.sync_copy(rows_v, out_hbm.at[pl.ds(b, ch)])
            return carry

        lax.fori_loop(0, nit, body, 0)

    return gk(table, idx)


def kernel(inputs, weight):
    n, c, h, wd = inputs.shape
    x = jnp.transpose(inputs, (0, 2, 3, 1))
    flat = x.reshape(-1, _D)
    nrows = flat.shape[0]
    x2 = jnp.sum(flat ** 2, axis=1, keepdims=True)
    w2 = jnp.sum(weight ** 2, axis=1)[None, :]
    colr = jnp.arange(_K)[None, :]
    seg_id = ((colr >= _B1).astype(jnp.int32)
              + (colr >= _B2).astype(jnp.int32))
    grid = (nrows // _BM,)
    enc, idx, dsel = pl.pallas_call(
        _vq_block,
        grid=grid,
        in_specs=[
            pl.BlockSpec((_BM, _D), lambda i: (i, 0)),
            pl.BlockSpec((_K, _D), lambda i: (0, 0)),
            pl.BlockSpec((_BM, 1), lambda i: (i, 0)),
            pl.BlockSpec((1, _K), lambda i: (0, 0)),
            pl.BlockSpec((1, _K), lambda i: (0, 0)),
        ],
        out_specs=[
            pl.BlockSpec((_BM, _K), lambda i: (i, 0)),
            pl.BlockSpec((_BM, 1), lambda i: (i, 0)),
            pl.BlockSpec((_BM, 1), lambda i: (i, 0)),
        ],
        out_shape=[
            jax.ShapeDtypeStruct((nrows, _K), jnp.float32),
            jax.ShapeDtypeStruct((nrows, 1), jnp.int32),
            jax.ShapeDtypeStruct((nrows, 1), jnp.float32),
        ],
    )(flat, weight.astype(jnp.bfloat16), x2, w2, seg_id)
    m = jnp.sum(dsel) / (nrows * _D)
    loss = m + _BETA * m
    wq = weight.astype(jnp.bfloat16).astype(jnp.float32)
    q = _sc_gather(wq, idx.reshape(-1))
    quantized_nchw = jnp.transpose(q.reshape(n, h, wd, c), (0, 3, 1, 2))
    return (loss, quantized_nchw, enc, idx)


# speedup vs baseline: 1.0256x; 1.0256x over previous
import functools

import jax
import jax.numpy as jnp
from jax import lax
from jax.experimental import pallas as pl
from jax.experimental.pallas import tpu as pltpu
from jax.experimental.pallas import tpu_sc as plsc

_K = 8192
_D = 256
_BM = 512
_B1 = 2736
_B2 = 5472
_A0 = 2688
_A1 = 5376
_BETA = 0.25


def _rowmin(a):
    return jnp.min(a, axis=1, keepdims=True)


def _vq_block(x_ref, w_ref, x2_ref, w2_ref, sid_ref, enc_ref, idx_ref,
              dsel_ref):
    x = x_ref[...]
    wb = w_ref[...]
    xb2 = x.astype(jnp.bfloat16) * jnp.bfloat16(2.0)
    t2 = jax.lax.dot_general(
        xb2, wb, (((1,), (1,)), ((), ())),
        preferred_element_type=jnp.float32)
    d = (x2_ref[...] + w2_ref[...]) - t2
    inf = jnp.float32(jnp.inf)
    sA = d[:, _A0:_A0 + 128]
    sB = d[:, _A1:_A1 + 128]
    lane = jax.lax.broadcasted_iota(jnp.int32, (_BM, 128), 1)
    mA = lane < (_B1 - _A0)
    mB = lane < (_B2 - _A1)
    m0 = jnp.minimum(_rowmin(d[:, :_A0]), _rowmin(jnp.where(mA, sA, inf)))
    m1 = jnp.minimum(
        jnp.minimum(_rowmin(jnp.where(mA, inf, sA)), _rowmin(d[:, _A0 + 128:_A1])),
        _rowmin(jnp.where(mB, sB, inf)))
    m2 = jnp.minimum(_rowmin(jnp.where(mB, inf, sB)), _rowmin(d[:, _A1 + 128:]))
    b0 = m0.astype(jnp.bfloat16).astype(jnp.float32)
    win1 = m1 < b0
    v1 = jnp.where(win1, m1, m0)
    b1 = v1.astype(jnp.bfloat16).astype(jnp.float32)
    win2 = m2 < b1
    dsel = jnp.where(win2, m2, v1)
    seg_sel = jnp.where(win2, 2, jnp.where(win1, 1, 0))
    cols = jax.lax.broadcasted_iota(jnp.int32, d.shape, 1)
    colsf = cols.astype(jnp.float32)
    pick = (d == dsel) & (sid_ref[...] == seg_sel)
    idxf = jnp.min(jnp.where(pick, colsf, jnp.float32(_K)), axis=1,
                   keepdims=True)
    idx = idxf.astype(jnp.int32)
    enc = (colsf == idxf).astype(jnp.float32)
    enc_ref[...] = enc
    idx_ref[...] = idx
    dsel_ref[...] = dsel


def _sc_gather(table, idx):
    info = plsc.get_sparse_core_info()
    nw = info.num_cores * info.num_subcores
    nb = idx.shape[0]
    dm = table.shape[1]
    b_per_w = nb // nw
    ch = 64
    nit = b_per_w // ch
    mesh = plsc.VectorSubcoreMesh(core_axis_name="c", subcore_axis_name="s")

    @functools.partial(
        pl.kernel, mesh=mesh,
        out_type=jax.ShapeDtypeStruct((nb, dm), jnp.float32),
        scratch_types=[
            pltpu.VMEM((ch,), jnp.int32),
            pltpu.VMEM((ch, dm), jnp.float32),
            pltpu.SemaphoreType.DMA,
        ],
    )
    def gk(table_hbm, idx_hbm, out_hbm, idx_v, rows_v, sem):
        wid = lax.axis_index("s") * info.num_cores + lax.axis_index("c")
        base = wid * b_per_w

        def body(i, carry):
            b = base + i * ch
            pltpu.sync_copy(idx_hbm.at[pl.ds(b, ch)], idx_v)
            pltpu.async_copy(table_hbm.at[idx_v], rows_v, sem).wait()
            pltpu.sync_copy(rows_v, out_hbm.at[pl.ds(b, ch)])
            return carry

        lax.fori_loop(0, nit, body, 0)

    return gk(table, idx)


def kernel(inputs, weight):
    n, c, h, wd = inputs.shape
    x = jnp.transpose(inputs, (0, 2, 3, 1))
    flat = x.reshape(-1, _D)
    nrows = flat.shape[0]
    x2 = jnp.sum(flat ** 2, axis=1, keepdims=True)
    w2 = jnp.sum(weight ** 2, axis=1)[None, :]
    colr = jnp.arange(_K)[None, :]
    seg_id = ((colr >= _B1).astype(jnp.int32)
              + (colr >= _B2).astype(jnp.int32))
    grid = (nrows // _BM,)
    enc, idx, dsel = pl.pallas_call(
        _vq_block,
        grid=grid,
        in_specs=[
            pl.BlockSpec((_BM, _D), lambda i: (i, 0)),
            pl.BlockSpec((_K, _D), lambda i: (0, 0)),
            pl.BlockSpec((_BM, 1), lambda i: (i, 0)),
            pl.BlockSpec((1, _K), lambda i: (0, 0)),
            pl.BlockSpec((1, _K), lambda i: (0, 0)),
        ],
        out_specs=[
            pl.BlockSpec((_BM, _K), lambda i: (i, 0)),
            pl.BlockSpec((_BM, 1), lambda i: (i, 0)),
            pl.BlockSpec((_BM, 1), lambda i: (i, 0)),
        ],
        out_shape=[
            jax.ShapeDtypeStruct((nrows, _K), jnp.float32),
            jax.ShapeDtypeStruct((nrows, 1), jnp.int32),
            jax.ShapeDtypeStruct((nrows, 1), jnp.float32),
        ],
    )(flat, weight.astype(jnp.bfloat16), x2, w2, seg_id)
    m = jnp.sum(dsel) / (nrows * _D)
    loss = m + _BETA * m
    wq = weight.astype(jnp.bfloat16).astype(jnp.float32)
    q = _sc_gather(wq, idx.reshape(-1))
    quantized_nchw = jnp.transpose(q.reshape(n, h, wd, c), (0, 3, 1, 2))
    return (loss, quantized_nchw, enc, idx)
